# baseline (speedup 1.0000x reference)
import functools

import jax
import jax.numpy as jnp
from jax import lax
from jax.experimental import pallas as pl
from jax.experimental.pallas import tpu as pltpu
from jax.experimental.pallas import tpu_sc as plsc

_NC, _NS = 2, 16
_NW = _NC * _NS


def _pool_body(T, Bn, n, d, c, hw2,
               x_ref, logits_ref,
               xm_ref, cls_ref, maxp_ref, cnt_ref):
    t = pl.program_id(0)

    @pl.when(t == 0)
    def _classify():
        logits = logits_ref[...]
        m = jnp.max(logits, axis=1, keepdims=True)
        s = jnp.sum(jnp.exp(logits - m), axis=1, keepdims=True)
        maxp_ref[...] = 1.0 / s
        iota_c = jax.lax.broadcasted_iota(jnp.int32, (n, c), 1)
        is_max = logits == m
        idx = jnp.min(jnp.where(is_max, iota_c, c), axis=1, keepdims=True)
        cls_ref[...] = idx.reshape(n // 128, 128)
        one_hot = (iota_c == idx).astype(jnp.float32)
        ones = jnp.ones((n, 1), dtype=jnp.float32)
        cnt_ref[...] = jax.lax.dot_general(
            one_hot, ones, (((0,), (0,)), ((), ())),
            preferred_element_type=jnp.float32)

    xb = x_ref[...]
    xm_ref[...] = jnp.sum(xb, axis=0) / float(hw2)


def _sc_seg_body(C, RW, xm_hbm, cls_hbm, seg_hbm,
                 idx_v, rows_v, part_v):
    cid = lax.axis_index("c")
    sid = lax.axis_index("s")
    wid = cid * _NS + sid
    base = wid * RW
    d = rows_v.shape[1]

    def _zrow(i, _):
        def _zcol(j, _):
            part_v[i, pl.ds(j * 16, 16)] = jnp.zeros((16,), jnp.float32)
            return 0
        return lax.fori_loop(0, d // 16, _zcol, 0)
    lax.fori_loop(0, C, _zrow, 0)

    pltpu.sync_copy(cls_hbm.at[pl.ds(base, RW)], idx_v.at[pl.ds(0, RW)])
    pltpu.sync_copy(xm_hbm.at[pl.ds(base, RW)], rows_v)

    def _row(r, _):
        cls_r = idx_v[pl.ds(r, 16)][0]

        def _col(j, _, cls_r=cls_r, r=r):
            chunk = rows_v[r, pl.ds(j * 16, 16)]
            plsc.addupdate(part_v.at[cls_r, pl.ds(j * 16, 16)], chunk)
            return 0
        lax.fori_loop(0, d // 16, _col, 0)
        return 0
    lax.fori_loop(0, RW, _row, 0)

    pltpu.sync_copy(part_v, seg_hbm.at[wid])


def _finish_body(n, d, c,
                 xm_ref, seg_ref, cnt_ref, proto_ref,
                 proto_new_ref, sim_ref):
    counts = cnt_ref[...]
    exist = counts > 0.0
    seg = jnp.sum(seg_ref[...], axis=0)
    local_mean = jnp.where(exist, seg / jnp.maximum(counts, 1.0), seg)
    proto = proto_ref[...]
    num = jnp.sum(proto * local_mean, axis=1, keepdims=True)
    den = (jnp.sqrt(jnp.sum(proto * proto, axis=1, keepdims=True)) *
           jnp.sqrt(jnp.sum(local_mean * local_mean, axis=1, keepdims=True)))
    mom = num / jnp.maximum(den, 1e-8)
    proto_new = jnp.where(exist, proto * mom + local_mean * (1.0 - mom), proto)
    proto_new_ref[...] = proto_new
    pn_norm = jnp.sqrt(jnp.sum(proto_new * proto_new, axis=1, keepdims=True))
    pn = proto_new / jnp.maximum(pn_norm, 1e-8)
    Bs = 128
    for i in range(n // Bs):
        xmb = xm_ref[pl.ds(i * Bs, Bs), :]
        norms = jnp.sqrt(jnp.sum(xmb * xmb, axis=1, keepdims=True))
        xn = xmb / jnp.maximum(norms, 1e-8)
        sim_ref[pl.ds(i * Bs, Bs), :] = jax.lax.dot_general(
            xn, pn, (((1,), (1,)), ((), ())),
            preferred_element_type=jnp.float32)


def kernel(x, class_logits, prototypes):
    n, d, hw, hw_ = x.shape
    hw2 = hw * hw_
    c = prototypes.shape[0]
    Bn = 64
    T = n // Bn
    RW = n // _NW

    xt = jnp.transpose(x, (2, 3, 0, 1)).reshape(hw2, n, d)

    pool = functools.partial(_pool_body, T, Bn, n, d, c, hw2)
    xm, cls2, maxp2, cnt = pl.pallas_call(
        pool,
        grid=(T,),
        in_specs=[
            pl.BlockSpec((hw2, Bn, d), lambda t: (0, t, 0)),
            pl.BlockSpec((n, c), lambda t: (0, 0)),
        ],
        out_specs=[
            pl.BlockSpec((Bn, d), lambda t: (t, 0)),
            pl.BlockSpec((n // 128, 128), lambda t: (0, 0)),
            pl.BlockSpec((n, 1), lambda t: (0, 0)),
            pl.BlockSpec((c, 1), lambda t: (0, 0)),
        ],
        out_shape=[
            jax.ShapeDtypeStruct((n, d), jnp.float32),
            jax.ShapeDtypeStruct((n // 128, 128), jnp.int32),
            jax.ShapeDtypeStruct((n, 1), jnp.float32),
            jax.ShapeDtypeStruct((c, 1), jnp.float32),
        ],
    )(xt, class_logits)

    cls_flat = cls2.reshape(n)

    sc_body = functools.partial(_sc_seg_body, c, RW)
    seg2 = pl.kernel(
        sc_body,
        out_type=jax.ShapeDtypeStruct((_NW, c, d), jnp.float32),
        mesh=plsc.VectorSubcoreMesh(core_axis_name="c", subcore_axis_name="s"),
        compiler_params=pltpu.CompilerParams(needs_layout_passes=False),
        scratch_types=[
            pltpu.VMEM((RW + 16, ), jnp.int32),
            pltpu.VMEM((RW, d), jnp.float32),
            pltpu.VMEM((c, d), jnp.float32),
        ],
    )(xm, cls_flat)

    finish = functools.partial(_finish_body, n, d, c)
    proto_new, sim = pl.pallas_call(
        finish,
        grid=(1,),
        in_specs=[
            pl.BlockSpec((n, d), lambda i: (0, 0)),
            pl.BlockSpec((_NW, c, d), lambda i: (0, 0, 0)),
            pl.BlockSpec((c, 1), lambda i: (0, 0)),
            pl.BlockSpec((c, d), lambda i: (0, 0)),
        ],
        out_specs=[
            pl.BlockSpec((c, d), lambda i: (0, 0)),
            pl.BlockSpec((n, c), lambda i: (0, 0)),
        ],
        out_shape=[
            jax.ShapeDtypeStruct((c, d), jnp.float32),
            jax.ShapeDtypeStruct((n, c), jnp.float32),
        ],
    )(xm, seg2, cnt, prototypes)

    return proto_new, sim, cls_flat, maxp2.reshape(n)


# speedup vs baseline: 1.4080x; 1.4080x over previous
import functools

import jax
import jax.numpy as jnp
from jax.experimental import pallas as pl
from jax.experimental.pallas import tpu as pltpu


def _body(T, Bn, n, d, c, hw2,
          x_ref, logits_ref, proto_ref,
          proto_new_ref, sim_ref, cls_ref, maxp_ref,
          xm_scr, oh_scr, seg_scr, cnt_scr):
    t = pl.program_id(0)

    @pl.when(t == 0)
    def _classify():
        logits = logits_ref[...]
        m = jnp.max(logits, axis=1, keepdims=True)
        s = jnp.sum(jnp.exp(logits - m), axis=1, keepdims=True)
        maxp_ref[...] = 1.0 / s
        iota_c = jax.lax.broadcasted_iota(jnp.int32, (n, c), 1)
        is_max = logits == m
        idx = jnp.min(jnp.where(is_max, iota_c, c), axis=1, keepdims=True)
        cls_ref[...] = idx
        one_hot = (iota_c == idx).astype(jnp.float32)
        oh_scr[...] = one_hot
        ones = jnp.ones((n, 1), dtype=jnp.float32)
        cnt_scr[...] = jax.lax.dot_general(
            one_hot, ones, (((0,), (0,)), ((), ())),
            preferred_element_type=jnp.float32,
            precision=jax.lax.Precision.HIGHEST)

    xb = x_ref[...]
    xm = jnp.sum(xb, axis=0) / float(hw2)
    norms = jnp.sqrt(jnp.sum(xm * xm, axis=1, keepdims=True))
    xm_scr[pl.ds(t * Bn, Bn), :] = xm / jnp.maximum(norms, 1e-8)

    oh_t = oh_scr[pl.ds(t * Bn, Bn), :]
    contrib = jax.lax.dot_general(
        oh_t, xm, (((0,), (0,)), ((), ())),
        preferred_element_type=jnp.float32)

    @pl.when(t == 0)
    def _seg_init():
        seg_scr[...] = contrib

    @pl.when(t > 0)
    def _seg_acc():
        seg_scr[...] = seg_scr[...] + contrib

    @pl.when(t == T - 1)
    def _finalize():
        counts = cnt_scr[...]
        exist = counts > 0.0
        seg = seg_scr[...]
        local_mean = jnp.where(exist, seg / jnp.maximum(counts, 1.0), seg)
        proto = proto_ref[...]
        num = jnp.sum(proto * local_mean, axis=1, keepdims=True)
        den = (jnp.sqrt(jnp.sum(proto * proto, axis=1, keepdims=True)) *
               jnp.sqrt(jnp.sum(local_mean * local_mean, axis=1, keepdims=True)))
        mom = num / jnp.maximum(den, 1e-8)
        proto_new = jnp.where(exist, proto * mom + local_mean * (1.0 - mom), proto)
        proto_new_ref[...] = proto_new
        pn_norm = jnp.sqrt(jnp.sum(proto_new * proto_new, axis=1, keepdims=True))
        pn = proto_new / jnp.maximum(pn_norm, 1e-8)
        Bs = 128
        for i in range(n // Bs):
            xn = xm_scr[pl.ds(i * Bs, Bs), :]
            sim_ref[pl.ds(i * Bs, Bs), :] = jax.lax.dot_general(
                xn, pn, (((1,), (1,)), ((), ())),
                preferred_element_type=jnp.float32)


def kernel(x, class_logits, prototypes):
    n, d, hw, hw_ = x.shape
    hw2 = hw * hw_
    c = prototypes.shape[0]
    Bn = 64
    T = n // Bn

    xt = jnp.transpose(x, (2, 3, 0, 1)).reshape(hw2, n, d)

    body = functools.partial(_body, T, Bn, n, d, c, hw2)
    proto_new, sim, cls2, maxp2 = pl.pallas_call(
        body,
        grid=(T,),
        in_specs=[
            pl.BlockSpec((hw2, Bn, d), lambda t: (0, t, 0)),
            pl.BlockSpec((n, c), lambda t: (0, 0)),
            pl.BlockSpec((c, d), lambda t: (0, 0)),
        ],
        out_specs=[
            pl.BlockSpec((c, d), lambda t: (0, 0)),
            pl.BlockSpec((n, c), lambda t: (0, 0)),
            pl.BlockSpec((n, 1), lambda t: (0, 0)),
            pl.BlockSpec((n, 1), lambda t: (0, 0)),
        ],
        out_shape=[
            jax.ShapeDtypeStruct((c, d), jnp.float32),
            jax.ShapeDtypeStruct((n, c), jnp.float32),
            jax.ShapeDtypeStruct((n, 1), jnp.int32),
            jax.ShapeDtypeStruct((n, 1), jnp.float32),
        ],
        scratch_shapes=[
            pltpu.VMEM((n, d), jnp.float32),
            pltpu.VMEM((n, c), jnp.float32),
            pltpu.VMEM((c, d), jnp.float32),
            pltpu.VMEM((c, 1), jnp.float32),
        ],
    )(xt, class_logits, prototypes)

    return proto_new, sim, cls2.reshape(n), maxp2.reshape(n)


# speedup vs baseline: 1.4733x; 1.0464x over previous
import functools

import jax
import jax.numpy as jnp
from jax.experimental import pallas as pl
from jax.experimental.pallas import tpu as pltpu


def _body(T, Bn, n, d, c, hw2,
          x_ref, logits_ref, proto_ref,
          proto_new_ref, sim_ref, cls_ref, maxp_ref,
          xm_scr, oh_scr, seg_scr, cnt_scr):
    t = pl.program_id(0)

    @pl.when(t == 0)
    def _classify():
        logits = logits_ref[...]
        m = jnp.max(logits, axis=1, keepdims=True)
        s = jnp.sum(jnp.exp(logits - m), axis=1, keepdims=True)
        maxp_ref[...] = 1.0 / s
        iota_c = jax.lax.broadcasted_iota(jnp.int32, (n, c), 1)
        is_max = logits == m
        idx = jnp.min(jnp.where(is_max, iota_c, c), axis=1, keepdims=True)
        cls_ref[...] = idx
        one_hot = (iota_c == idx).astype(jnp.float32)
        oh_scr[...] = one_hot
        ones = jnp.ones((n, 1), dtype=jnp.float32)
        cnt_scr[...] = jax.lax.dot_general(
            one_hot, ones, (((0,), (0,)), ((), ())),
            preferred_element_type=jnp.float32,
            precision=jax.lax.Precision.HIGHEST)

    xb = x_ref[...]
    xm = jnp.sum(xb, axis=0) / float(hw2)
    xm_scr[pl.ds(t * Bn, Bn), :] = xm

    oh_t = oh_scr[pl.ds(t * Bn, Bn), :]
    contrib = jax.lax.dot_general(
        oh_t, xm, (((0,), (0,)), ((), ())),
        preferred_element_type=jnp.float32)

    @pl.when(t == 0)
    def _seg_init():
        seg_scr[...] = contrib

    @pl.when(t > 0)
    def _seg_acc():
        seg_scr[...] = seg_scr[...] + contrib

    @pl.when(t == T - 1)
    def _finalize():
        counts = cnt_scr[...]
        exist = counts > 0.0
        seg = seg_scr[...]
        local_mean = jnp.where(exist, seg / jnp.maximum(counts, 1.0), seg)
        proto = proto_ref[...]
        num = jnp.sum(proto * local_mean, axis=1, keepdims=True)
        den = (jnp.sqrt(jnp.sum(proto * proto, axis=1, keepdims=True)) *
               jnp.sqrt(jnp.sum(local_mean * local_mean, axis=1, keepdims=True)))
        mom = num / jnp.maximum(den, 1e-8)
        proto_new = jnp.where(exist, proto * mom + local_mean * (1.0 - mom), proto)
        proto_new_ref[...] = proto_new
        pn_norm = jnp.sqrt(jnp.sum(proto_new * proto_new, axis=1, keepdims=True))
        pn = proto_new / jnp.maximum(pn_norm, 1e-8)
        Bs = 128
        for i in range(n // Bs):
            xmb = xm_scr[pl.ds(i * Bs, Bs), :]
            norms = jnp.sqrt(jnp.sum(xmb * xmb, axis=1, keepdims=True))
            xn = xmb / jnp.maximum(norms, 1e-8)
            sim_ref[pl.ds(i * Bs, Bs), :] = jax.lax.dot_general(
                xn, pn, (((1,), (1,)), ((), ())),
                preferred_element_type=jnp.float32)


def kernel(x, class_logits, prototypes):
    n, d, hw, hw_ = x.shape
    hw2 = hw * hw_
    c = prototypes.shape[0]
    Bn = 64
    T = n // Bn

    xt = jnp.transpose(x, (2, 3, 0, 1)).reshape(hw2, n, d)

    body = functools.partial(_body, T, Bn, n, d, c, hw2)
    proto_new, sim, cls2, maxp2 = pl.pallas_call(
        body,
        grid=(T,),
        in_specs=[
            pl.BlockSpec((hw2, Bn, d), lambda t: (0, t, 0)),
            pl.BlockSpec((n, c), lambda t: (0, 0)),
            pl.BlockSpec((c, d), lambda t: (0, 0)),
        ],
        out_specs=[
            pl.BlockSpec((c, d), lambda t: (0, 0)),
            pl.BlockSpec((n, c), lambda t: (0, 0)),
            pl.BlockSpec((n, 1), lambda t: (0, 0)),
            pl.BlockSpec((n, 1), lambda t: (0, 0)),
        ],
        out_shape=[
            jax.ShapeDtypeStruct((c, d), jnp.float32),
            jax.ShapeDtypeStruct((n, c), jnp.float32),
            jax.ShapeDtypeStruct((n, 1), jnp.int32),
            jax.ShapeDtypeStruct((n, 1), jnp.float32),
        ],
        scratch_shapes=[
            pltpu.VMEM((n, d), jnp.float32),
            pltpu.VMEM((n, c), jnp.float32),
            pltpu.VMEM((c, d), jnp.float32),
            pltpu.VMEM((c, 1), jnp.float32),
        ],
    )(xt, class_logits, prototypes)

    return proto_new, sim, cls2.reshape(n), maxp2.reshape(n)
